# Initial kernel scaffold; baseline (speedup 1.0000x reference)
import jax
import jax.numpy as jnp
from jax.experimental import pallas as pl


def kernel(query, keys):
    raise NotImplementedError("write your pallas kernel here")


# speedup vs baseline: 13.2199x; 13.2199x over previous
import jax
import jax.numpy as jnp
from jax import lax
from jax.experimental import pallas as pl
from jax.experimental.pallas import tpu as pltpu

_B, _D, _H, _W = 8, 256, 32, 32
_M = 1024
_N = _B * _H * _W
_T = 256
_NT = _N // _T
_TPB = (_H * _W) // _T
_NEG = -1e30


def _norm_rows(q):
    n2 = jnp.sum(q * q, axis=1, keepdims=True)
    return q * lax.rsqrt(jnp.maximum(n2, 1e-24))


def _stats_kernel(qf_ref, keys_ref, cmax_ref, csum_ref):
    i = pl.program_id(0)
    qn = _norm_rows(qf_ref[...])
    score = lax.dot_general(qn, keys_ref[...], (((1,), (1,)), ((), ())),
                            preferred_element_type=jnp.float32)
    tmax = jnp.max(score, axis=0, keepdims=True)

    @pl.when(i == 0)
    def _():
        cmax_ref[...] = jnp.full((1, _M), _NEG, jnp.float32)
        csum_ref[...] = jnp.zeros((1, _M), jnp.float32)

    m_old = cmax_ref[...]
    m_new = jnp.maximum(m_old, tmax)
    csum_ref[...] = (csum_ref[...] * jnp.exp(m_old - m_new)
                     + jnp.sum(jnp.exp(score - m_new), axis=0, keepdims=True))
    cmax_ref[...] = m_new


def _main_kernel(qf_ref, keys_ref, cmax_ref, csum_ref,
                 sm_ref, sq_ref, qcat_ref, g_ref, s_ref, upd_ref,
                 gscr, sscr, uscr):
    i = pl.program_id(0)
    q = qf_ref[...]
    keys = keys_ref[...]
    qn = _norm_rows(q)
    qn2 = jnp.sum(qn * qn, axis=1, keepdims=True)
    qsum = jnp.sum(qn, axis=1, keepdims=True)
    score = lax.dot_general(qn, keys, (((1,), (1,)), ((), ())),
                            preferred_element_type=jnp.float32)

    rmax = jnp.max(score, axis=1, keepdims=True)
    e = jnp.exp(score - rmax)
    sm = e / jnp.sum(e, axis=1, keepdims=True)
    sm_ref[...] = sm

    cmaxb = cmax_ref[...]
    sq_ref[...] = jnp.exp(score - cmaxb) / csum_ref[...]

    cmem = lax.dot_general(sm, keys, (((1,), (0,)), ((), ())),
                           preferred_element_type=jnp.float32)
    qcat_ref[:, :_D] = qn
    qcat_ref[:, _D:] = cmem

    col = lax.broadcasted_iota(jnp.int32, (_T, _M), 1)
    gi = jnp.min(jnp.where(score >= rmax, col, _M), axis=1, keepdims=True)
    oh1 = col == gi
    masked = jnp.where(oh1, _NEG, score)
    m2 = jnp.max(masked, axis=1, keepdims=True)
    gi2 = jnp.min(jnp.where(masked >= m2, col, _M), axis=1, keepdims=True)
    oh2 = col == gi2

    ones_row = jnp.ones((1, _D), jnp.float32)
    ksum = lax.dot_general(ones_row, keys, (((1,), (1,)), ((), ())),
                           preferred_element_type=jnp.float32)
    kn2 = lax.dot_general(ones_row, keys * keys, (((1,), (1,)), ((), ())),
                          preferred_element_type=jnp.float32)

    def gath(vec_row, oh):
        return jnp.sum(jnp.where(oh, vec_row, 0.0), axis=1, keepdims=True)

    cmax_g = gath(cmaxb, oh1)
    kn2_g, ksum_g = gath(kn2, oh1), gath(ksum, oh1)
    kn2_g2, ksum_g2 = gath(kn2, oh2), gath(ksum, oh2)

    @pl.when(i == 0)
    def _():
        gscr[...] = jnp.zeros((_B, 1), jnp.float32)
        sscr[...] = jnp.zeros((_B, 1), jnp.float32)
        uscr[...] = jnp.zeros((_M, _D), jnp.float32)

    bid = i // _TPB
    boh = lax.broadcasted_iota(jnp.int32, (_B, 1), 0) == bid

    d1sq = qn2 - 2.0 * rmax + kn2_g
    gscr[...] += jnp.where(boh, jnp.sum(d1sq) / (_H * _W * _D * 1.0), 0.0)

    eps = 1e-6
    d2sq = qn2 - 2.0 * m2 + kn2_g2
    dp = jnp.sqrt(jnp.maximum(d1sq + 2 * eps * (qsum - ksum_g) + _D * eps * eps, 0.0))
    dn = jnp.sqrt(jnp.maximum(d2sq + 2 * eps * (qsum - ksum_g2) + _D * eps * eps, 0.0))
    s_row = jnp.maximum(dp - dn + 1.0, 0.0)
    sscr[...] += jnp.where(boh, jnp.sum(s_row) / (_H * _W * 1.0), 0.0)

    wgt = jnp.exp(rmax - cmax_g)
    wm = jnp.where(oh1, wgt, 0.0)
    uscr[...] += lax.dot_general(wm, qn, (((0,), (0,)), ((), ())),
                                 preferred_element_type=jnp.float32)

    @pl.when(i == _NT - 1)
    def _():
        g_ref[...] = gscr[...]
        s_ref[...] = sscr[...]
        upd_ref[...] = _norm_rows(uscr[...] + keys)


def kernel(query, keys):
    qf = jnp.transpose(query, (0, 2, 3, 1)).reshape(_N, _D)
    f32 = jnp.float32

    cmax, csum = pl.pallas_call(
        _stats_kernel,
        grid=(_NT,),
        in_specs=[
            pl.BlockSpec((_T, _D), lambda i: (i, 0)),
            pl.BlockSpec((_M, _D), lambda i: (0, 0)),
        ],
        out_specs=[
            pl.BlockSpec((1, _M), lambda i: (0, 0)),
            pl.BlockSpec((1, _M), lambda i: (0, 0)),
        ],
        out_shape=[
            jax.ShapeDtypeStruct((1, _M), f32),
            jax.ShapeDtypeStruct((1, _M), f32),
        ],
    )(qf, keys)

    sm, sq, qcat, g_loss, s_loss, upd = pl.pallas_call(
        _main_kernel,
        grid=(_NT,),
        in_specs=[
            pl.BlockSpec((_T, _D), lambda i: (i, 0)),
            pl.BlockSpec((_M, _D), lambda i: (0, 0)),
            pl.BlockSpec((1, _M), lambda i: (0, 0)),
            pl.BlockSpec((1, _M), lambda i: (0, 0)),
        ],
        out_specs=[
            pl.BlockSpec((_T, _M), lambda i: (i, 0)),
            pl.BlockSpec((_T, _M), lambda i: (i, 0)),
            pl.BlockSpec((_T, 2 * _D), lambda i: (i, 0)),
            pl.BlockSpec((_B, 1), lambda i: (0, 0)),
            pl.BlockSpec((_B, 1), lambda i: (0, 0)),
            pl.BlockSpec((_M, _D), lambda i: (0, 0)),
        ],
        out_shape=[
            jax.ShapeDtypeStruct((_N, _M), f32),
            jax.ShapeDtypeStruct((_N, _M), f32),
            jax.ShapeDtypeStruct((_N, 2 * _D), f32),
            jax.ShapeDtypeStruct((_B, 1), f32),
            jax.ShapeDtypeStruct((_B, 1), f32),
            jax.ShapeDtypeStruct((_M, _D), f32),
        ],
        scratch_shapes=[
            pltpu.VMEM((_B, 1), f32),
            pltpu.VMEM((_B, 1), f32),
            pltpu.VMEM((_M, _D), f32),
        ],
    )(qf, keys, cmax, csum)

    uq = qcat.reshape(_B, _H, _W, 2 * _D).transpose(0, 3, 1, 2)
    return (uq, upd, sq, sm, g_loss, s_loss)
